# Initial kernel scaffold; baseline (speedup 1.0000x reference)
import jax
import jax.numpy as jnp
from jax.experimental import pallas as pl


def kernel(x, wih, whh, bias, fc_w_t, fc_b):
    raise NotImplementedError("write your pallas kernel here")


# speedup vs baseline: 1.2318x; 1.2318x over previous
import functools

import jax
import jax.numpy as jnp
from jax.experimental import pallas as pl
from jax.experimental.pallas import tpu as pltpu


def _make_body(T, Bc, H, Din, Op):
    TB = T * Bc

    def body(x_ref, wih0_ref, whh0_ref, w1_ref, bias_ref, fcw_ref, fcb_ref,
             out_ref, hcn_ref, g0_sc, h1_sc):

        g0_sc[...] = (
            jnp.dot(x_ref[...], wih0_ref[...], preferred_element_type=jnp.float32)
            + bias_ref[0])

        whh0 = whh0_ref[...]
        w1 = w1_ref[...]
        b1 = bias_ref[1]

        def cell(g, c):
            ifo = jax.nn.sigmoid(g[:, :3 * H])
            g_t = jnp.tanh(g[:, 3 * H:])
            i_g = ifo[:, :H]
            f_g = ifo[:, H:2 * H]
            o_g = ifo[:, 2 * H:]
            c_new = f_g * c + i_g * g_t
            h_new = o_g * jnp.tanh(c_new)
            return h_new, c_new

        def l0_step(t, h0, c0):
            g = g0_sc[pl.ds(t * Bc, Bc), :] + jnp.dot(
                h0.astype(jnp.bfloat16), whh0, preferred_element_type=jnp.float32)
            return cell(g, c0)

        def l1_step(t, x1, h1, c1):
            a = jnp.concatenate([x1, h1.astype(jnp.bfloat16)], axis=1)
            g = jnp.dot(a, w1, preferred_element_type=jnp.float32) + b1
            h_new, c_new = cell(g, c1)
            h1_sc[pl.ds(t * Bc, Bc), :] = h_new.astype(jnp.bfloat16)
            return h_new, c_new

        h0, c0 = cell(g0_sc[pl.ds(0, Bc), :], jnp.zeros((Bc, H), jnp.float32))

        def l1_first(x1):
            g = jnp.dot(x1, w1[:H, :], preferred_element_type=jnp.float32) + b1
            h_new, c_new = cell(g, jnp.zeros((Bc, H), jnp.float32))
            h1_sc[pl.ds(0, Bc), :] = h_new.astype(jnp.bfloat16)
            return h_new, c_new

        h1 = c1 = None
        for i in range(1, T):
            h0_prev = h0.astype(jnp.bfloat16)
            h0, c0 = l0_step(i, h0, c0)
            if i == 1:
                h1, c1 = l1_first(h0_prev)
            else:
                h1, c1 = l1_step(i - 1, h0_prev, h1, c1)
        h1, c1 = l1_step(T - 1, h0.astype(jnp.bfloat16), h1, c1)

        hcn_ref[0] = jnp.concatenate([h0, c0], axis=-1)
        hcn_ref[1] = jnp.concatenate([h1, c1], axis=-1)

        out_ref[...] = (
            jnp.dot(h1_sc[...], fcw_ref[...], preferred_element_type=jnp.float32)
            + fcb_ref[...])

    return body


@functools.partial(jax.jit, static_argnames=("T", "Bc", "H", "Din", "Op", "NC"))
def _forward(x_tm, wih0, whh0, w1, bias, fcw, fcb, *, T, Bc, H, Din, Op, NC):
    TB = T * Bc
    body = _make_body(T, Bc, H, Din, Op)

    vmem_bytes = (
        TB * Din * 2
        + TB * Op * 4
        + TB * 4 * H * 4
        + TB * H * 2
        + Din * 4 * H * 2
        + H * 4 * H * 2
        + 2 * H * 4 * H * 2
        + 2 * 4 * H * 4
        + H * Op * 2 + Op * 4
        + 2 * Bc * 2 * H * 4)
    vmem_limit = int(min(2 * vmem_bytes + (2 << 20), 64 << 20))

    return pl.pallas_call(
        body,
        out_shape=(
            jax.ShapeDtypeStruct((NC, TB, Op), jnp.float32),
            jax.ShapeDtypeStruct((NC, 2, Bc, 2 * H), jnp.float32),
        ),
        grid=(NC,),
        in_specs=[
            pl.BlockSpec((None, TB, Din), lambda c: (c, 0, 0)),
            pl.BlockSpec((Din, 4 * H), lambda c: (0, 0)),
            pl.BlockSpec((H, 4 * H), lambda c: (0, 0)),
            pl.BlockSpec((2 * H, 4 * H), lambda c: (0, 0)),
            pl.BlockSpec((2, 1, 4 * H), lambda c: (0, 0, 0)),
            pl.BlockSpec((H, Op), lambda c: (0, 0)),
            pl.BlockSpec((1, Op), lambda c: (0, 0)),
        ],
        out_specs=(
            pl.BlockSpec((None, TB, Op), lambda c: (c, 0, 0)),
            pl.BlockSpec((None, 2, Bc, 2 * H), lambda c: (c, 0, 0, 0)),
        ),
        scratch_shapes=[
            pltpu.VMEM((TB, 4 * H), jnp.float32),
            pltpu.VMEM((TB, H), jnp.bfloat16),
        ],
        compiler_params=pltpu.CompilerParams(
            dimension_semantics=("arbitrary",),
            vmem_limit_bytes=vmem_limit),
    )(x_tm, wih0, whh0, w1, bias, fcw, fcb)


def kernel(x, wih, whh, bias, fc_w_t, fc_b):
    B, T, D0 = x.shape
    L, Din, fourH = wih.shape
    H = fourH // 4
    Op = fc_w_t.shape[1]
    O = fc_b.shape[1]

    NC = 1
    Bc = max(16, -(-B // NC // 16) * 16)
    Bp = NC * Bc
    TB = T * Bc

    x_p = jnp.pad(x, ((0, Bp - B), (0, 0), (0, Din - D0)))
    x_tm = jnp.transpose(x_p.reshape(NC, Bc, T, Din), (0, 2, 1, 3))
    x_tm = x_tm.reshape(NC, TB, Din).astype(jnp.bfloat16)

    w1 = jnp.concatenate([wih[1, :H, :], whh[1]], axis=0)

    fcb_p = jnp.pad(fc_b, ((0, 0), (0, Op - O)))

    out2d, hcn = _forward(
        x_tm, wih[0], whh[0], w1, bias, fc_w_t, fcb_p,
        T=T, Bc=Bc, H=H, Din=Din, Op=Op, NC=NC)

    out4 = out2d.reshape(NC, T, Bc, Op)
    out = jnp.transpose(out4, (0, 2, 1, 3)).reshape(Bp, T, Op)[:B, :, :O]

    hcn_l = jnp.transpose(hcn, (1, 0, 2, 3)).reshape(L, Bp, 2 * H)[:, :B, :]
    hn = hcn_l[:, :, :H]
    cn = hcn_l[:, :, H:]
    return out, (hn, cn)
